# Initial kernel scaffold; baseline (speedup 1.0000x reference)
import jax
import jax.numpy as jnp
from jax.experimental import pallas as pl


def kernel(x, table):
    raise NotImplementedError("write your pallas kernel here")


# speedup vs baseline: 1.0984x; 1.0984x over previous
import functools

import jax
import jax.numpy as jnp
from jax import lax
from jax.experimental import pallas as pl
from jax.experimental.pallas import tpu as pltpu
from jax.experimental.pallas import tpu_sc as plsc

_NC = 2
_NS = 16
_NW = _NC * _NS
_LANES = 16
_CHUNK = 1024


@functools.lru_cache(maxsize=None)
def _make(B: int, V: int, D: int):
    assert B % (_NW * _LANES) == 0
    b_per_w = B // _NW
    assert b_per_w % _CHUNK == 0
    n_chunks = b_per_w // _CHUNK

    mesh = plsc.VectorSubcoreMesh(core_axis_name="c", subcore_axis_name="s")

    @functools.partial(
        pl.kernel,
        mesh=mesh,
        out_type=jax.ShapeDtypeStruct((B, D), jnp.float32),
        scratch_types=[
            pltpu.VMEM((b_per_w,), jnp.int32),
            pltpu.VMEM((_CHUNK, D), jnp.float32),
            pltpu.SemaphoreType.DMA,
        ],
        compiler_params=pltpu.CompilerParams(use_tc_tiling_on_sc=False),
    )
    def k(idx_hbm, table_hbm, out_hbm, idx_v, rows_v, dma_sem):
        wid = lax.axis_index("s") * _NC + lax.axis_index("c")
        base = wid * b_per_w

        pltpu.sync_copy(idx_hbm.at[pl.ds(base, b_per_w)], idx_v)

        def add_one(i, carry):
            sl = pl.ds(i * _LANES, _LANES)
            idx_v[sl] = idx_v[sl] + 1
            return carry

        lax.fori_loop(0, b_per_w // _LANES, add_one, 0)

        for g in range(n_chunks):
            sl = pl.ds(g * _CHUNK, _CHUNK)
            pltpu.async_copy(table_hbm.at[idx_v.at[sl]], rows_v, dma_sem).wait()
            pltpu.sync_copy(rows_v, out_hbm.at[pl.ds(base + g * _CHUNK, _CHUNK)])

    return k


def kernel(x, table):
    B = x.shape[0] * x.shape[1]
    V, D = table.shape
    idx = x.reshape(B).astype(jnp.int32)
    out = _make(B, V, D)(idx, table)
    return out.reshape(x.shape[0], x.shape[1], D)


# speedup vs baseline: 1.1134x; 1.0137x over previous
import functools

import jax
import jax.numpy as jnp
from jax import lax
from jax.experimental import pallas as pl
from jax.experimental.pallas import tpu as pltpu
from jax.experimental.pallas import tpu_sc as plsc

_NC = 2
_NS = 16
_NW = _NC * _NS
_LANES = 16
_CHUNK = 1024
_NBUF = 3


@functools.lru_cache(maxsize=None)
def _make(B: int, V: int, D: int):
    assert B % (_NW * _LANES) == 0
    b_per_w = B // _NW
    assert b_per_w % _CHUNK == 0
    n_chunks = b_per_w // _CHUNK

    mesh = plsc.VectorSubcoreMesh(core_axis_name="c", subcore_axis_name="s")

    @functools.partial(
        pl.kernel,
        mesh=mesh,
        out_type=jax.ShapeDtypeStruct((B, D), jnp.float32),
        scratch_types=[
            pltpu.VMEM((b_per_w,), jnp.int32),
            pltpu.VMEM((_CHUNK, D), jnp.float32),
            pltpu.VMEM((_CHUNK, D), jnp.float32),
            pltpu.VMEM((_CHUNK, D), jnp.float32),
            pltpu.SemaphoreType.DMA,
            pltpu.SemaphoreType.DMA,
        ],
        compiler_params=pltpu.CompilerParams(use_tc_tiling_on_sc=False),
    )
    def k(idx_hbm, table_hbm, out_hbm, idx_v, rows0, rows1, rows2, gsem, wsem):
        rows = (rows0, rows1, rows2)
        wid = lax.axis_index("s") * _NC + lax.axis_index("c")
        base = wid * b_per_w

        pltpu.sync_copy(idx_hbm.at[pl.ds(base, b_per_w)], idx_v)

        def add_one_chunk(g):
            def body(i, carry):
                sl = pl.ds(g * _CHUNK + i * _LANES, _LANES)
                idx_v[sl] = idx_v[sl] + 1
                return carry

            lax.fori_loop(0, _CHUNK // _LANES, body, 0)

        gathers = [None] * n_chunks
        writes = [None] * n_chunks
        for g in range(n_chunks):
            b = g % _NBUF
            if g >= _NBUF:
                writes[g - _NBUF].wait()
            add_one_chunk(g)
            gathers[g] = pltpu.async_copy(
                table_hbm.at[idx_v.at[pl.ds(g * _CHUNK, _CHUNK)]], rows[b], gsem)
            if g >= 1:
                gathers[g - 1].wait()
                writes[g - 1] = pltpu.async_copy(
                    rows[(g - 1) % _NBUF],
                    out_hbm.at[pl.ds(base + (g - 1) * _CHUNK, _CHUNK)], wsem)
        g = n_chunks - 1
        gathers[g].wait()
        writes[g] = pltpu.async_copy(
            rows[g % _NBUF], out_hbm.at[pl.ds(base + g * _CHUNK, _CHUNK)], wsem)
        for g in range(n_chunks - _NBUF, n_chunks):
            writes[g].wait()

    return k


def kernel(x, table):
    B = x.shape[0] * x.shape[1]
    V, D = table.shape
    idx = x.reshape(B).astype(jnp.int32)
    out = _make(B, V, D)(idx, table)
    return out.reshape(x.shape[0], x.shape[1], D)


# speedup vs baseline: 1.5200x; 1.3652x over previous
import functools

import jax
import jax.numpy as jnp
from jax import lax
from jax.experimental import pallas as pl
from jax.experimental.pallas import tpu as pltpu
from jax.experimental.pallas import tpu_sc as plsc

_NC = 2
_NS = 16
_NW = _NC * _NS
_L = 16

_I = 50
_BB = 16384
_D = 32
_CHUNK = 512
_BC = _BB // _CHUNK
_N_ITEMS = _I * _BC
_PER_W = _N_ITEMS // _NW
_FT = _D // 8
_BT = _CHUNK // 128


@functools.lru_cache(maxsize=None)
def _make(V: int):
    mesh = plsc.VectorSubcoreMesh(core_axis_name="c", subcore_axis_name="s")

    @functools.partial(
        pl.kernel,
        mesh=mesh,
        out_type=jax.ShapeDtypeStruct((_I, _FT, _BB // 128, 8, 128),
                                      jnp.float32),
        scratch_types=[
            pltpu.VMEM((2, _CHUNK), jnp.int32),
            pltpu.VMEM((2, _CHUNK, _D), jnp.float32),
            pltpu.VMEM((2, _D, _CHUNK), jnp.float32),
            pltpu.SemaphoreType.DMA,
            pltpu.SemaphoreType.DMA,
            pltpu.SemaphoreType.DMA,
            pltpu.SemaphoreType.DMA,
            pltpu.SemaphoreType.DMA,
        ],
        compiler_params=pltpu.CompilerParams(use_tc_tiling_on_sc=False,
                                             needs_layout_passes=False),
    )
    def k(xt_hbm, tbl_hbm, out_hbm, idx_s, rows_s, tbuf_s,
          isem, gsem0, gsem1, wsem0, wsem1):
        wid = lax.axis_index("s") * _NC + lax.axis_index("c")
        g0 = wid * _PER_W

        def item(t):
            g = g0 + t
            return g // _BC, g % _BC

        def start_idx_dma(t, buf):
            i, bc = item(t)
            return pltpu.async_copy(
                xt_hbm.at[i, pl.ds(bc * _CHUNK, _CHUNK)], idx_s.at[buf], isem)

        def add_one(buf):
            def body(j, carry):
                sl = pl.ds(j * _L, _L)
                idx_s[buf, sl] = idx_s[buf, sl] + 1
                return carry
            lax.fori_loop(0, _CHUNK // _L, body, 0)

        def start_gather(buf, gsem):
            return pltpu.async_copy(
                tbl_hbm.at[idx_s.at[buf]], rows_s.at[buf], gsem)

        start_idx_dma(0, 0).wait()
        add_one(0)
        start_gather(0, gsem0)
        start_idx_dma(1, 1)

        iota16 = lax.iota(jnp.int32, _L)

        def one_item(t, buf, gsem_cur, gsem_nxt, wsem_cur):
            i_t, bc_t = item(t)

            @pl.when(t < _PER_W - 1)
            def _():
                pltpu.make_async_copy(
                    xt_hbm.at[0, pl.ds(0, _CHUNK)], idx_s.at[1 - buf],
                    isem).wait()
                add_one(1 - buf)
                start_gather(1 - buf, gsem_nxt)

            pltpu.make_async_copy(
                tbl_hbm.at[pl.ds(0, _CHUNK)], rows_s.at[buf],
                gsem_cur).wait()

            @pl.when(t >= 2)
            def _():
                def wdrain(j, carry):
                    pltpu.make_async_copy(
                        out_hbm.at[0, 0, 0],
                        tbuf_s.at[buf, pl.ds(0, 8), pl.ds(0, 128)],
                        wsem_cur).wait()
                    return carry
                lax.fori_loop(0, _FT * _BT, wdrain, 0)

            for f in range(_D):
                fvec = jnp.full((_L,), f, jnp.int32)

                def tbody(j, carry, f=f, fvec=fvec):
                    b0 = j * _L
                    vals = plsc.load_gather(rows_s.at[buf],
                                            [b0 + iota16, fvec])
                    tbuf_s[buf, f, pl.ds(b0, _L)] = vals
                    return carry
                lax.fori_loop(0, _CHUNK // _L, tbody, 0)

            for fg in range(_FT):
                for btl in range(_BT):
                    pltpu.async_copy(
                        tbuf_s.at[buf, pl.ds(8 * fg, 8),
                                  pl.ds(128 * btl, 128)],
                        out_hbm.at[i_t, fg, bc_t * _BT + btl], wsem_cur)

            @pl.when(t < _PER_W - 2)
            def _():
                start_idx_dma(t + 2, buf)

        def body(u, carry):
            one_item(2 * u, 0, gsem0, gsem1, wsem0)
            one_item(2 * u + 1, 1, gsem1, gsem0, wsem1)
            return carry

        lax.fori_loop(0, _PER_W // 2, body, 0)

        def wdrain2(j, carry):
            pltpu.make_async_copy(
                out_hbm.at[0, 0, 0],
                tbuf_s.at[0, pl.ds(0, 8), pl.ds(0, 128)], wsem0).wait()
            pltpu.make_async_copy(
                out_hbm.at[0, 0, 0],
                tbuf_s.at[1, pl.ds(0, 8), pl.ds(0, 128)], wsem1).wait()
            return carry
        lax.fori_loop(0, _FT * _BT, wdrain2, 0)

    return k


def kernel(x, table):
    V, D = table.shape
    xt = jnp.transpose(x).astype(jnp.int32)
    out5 = _make(V)(xt, table)
    return out5.transpose(2, 4, 0, 1, 3).reshape(_BB, _I, D)


# speedup vs baseline: 2.3587x; 1.5518x over previous
import functools

import jax
import jax.numpy as jnp
from jax import lax
from jax.experimental import pallas as pl
from jax.experimental.pallas import tpu as pltpu
from jax.experimental.pallas import tpu_sc as plsc

_NC = 2
_NS = 16
_NW = _NC * _NS
_L = 16

_I = 50
_BB = 16384
_D = 32
_CHUNK = 512
_BC = _BB // _CHUNK
_N_ITEMS = _I * _BC
_PER_W = _N_ITEMS // _NW
_FT = _D // 8
_BT = _CHUNK // 128


@functools.lru_cache(maxsize=None)
def _make(V: int):
    mesh = plsc.VectorSubcoreMesh(core_axis_name="c", subcore_axis_name="s")

    @functools.partial(
        pl.kernel,
        mesh=mesh,
        out_type=jax.ShapeDtypeStruct((_I, _FT, _BB // 128, 8, 128),
                                      jnp.float32),
        scratch_types=[
            pltpu.VMEM((2, _CHUNK), jnp.int32),
            pltpu.VMEM((2, _CHUNK, _D), jnp.float32),
            pltpu.VMEM((2, _D, _CHUNK + 1), jnp.float32),
            pltpu.SemaphoreType.DMA,
            pltpu.SemaphoreType.DMA,
            pltpu.SemaphoreType.DMA,
            pltpu.SemaphoreType.DMA,
            pltpu.SemaphoreType.DMA,
        ],
        compiler_params=pltpu.CompilerParams(use_tc_tiling_on_sc=False,
                                             needs_layout_passes=False),
    )
    def k(xt_hbm, tbl_hbm, out_hbm, idx_s, rows_s, tbuf_s,
          isem, gsem0, gsem1, wsem0, wsem1):
        wid = lax.axis_index("s") * _NC + lax.axis_index("c")
        g0 = wid * _PER_W

        def item(t):
            g = g0 + t
            return g // _BC, g % _BC

        def start_idx_dma(t, buf):
            i, bc = item(t)
            return pltpu.async_copy(
                xt_hbm.at[i, pl.ds(bc * _CHUNK, _CHUNK)], idx_s.at[buf], isem)

        def add_one(buf):
            def body(j, carry):
                sl = pl.ds(j * _L, _L)
                idx_s[buf, sl] = idx_s[buf, sl] + 1
                return carry
            lax.fori_loop(0, _CHUNK // _L, body, 0)

        def start_gather(buf, gsem):
            return pltpu.async_copy(
                tbl_hbm.at[idx_s.at[buf]], rows_s.at[buf], gsem)

        start_idx_dma(0, 0).wait()
        add_one(0)
        start_gather(0, gsem0)
        start_idx_dma(1, 1)

        iota16 = lax.iota(jnp.int32, _L)

        def one_item(t, buf, gsem_cur, gsem_nxt, wsem_cur):
            i_t, bc_t = item(t)

            @pl.when(t < _PER_W - 1)
            def _():
                pltpu.make_async_copy(
                    xt_hbm.at[0, pl.ds(0, _CHUNK)], idx_s.at[1 - buf],
                    isem).wait()
                add_one(1 - buf)
                start_gather(1 - buf, gsem_nxt)

            pltpu.make_async_copy(
                tbl_hbm.at[pl.ds(0, _CHUNK)], rows_s.at[buf],
                gsem_cur).wait()

            @pl.when(t >= 2)
            def _():
                def wdrain(j, carry):
                    pltpu.make_async_copy(
                        out_hbm.at[0, 0, 0],
                        tbuf_s.at[buf, pl.ds(0, 8), pl.ds(0, 128)],
                        wsem_cur).wait()
                    return carry
                lax.fori_loop(0, _FT * _BT, wdrain, 0)

            def tbody(bp, carry):
                for f0 in range(0, _D, _L):
                    vals = rows_s[buf, bp, pl.ds(f0, _L)]
                    plsc.store_scatter(
                        tbuf_s.at[buf],
                        [f0 + iota16, jnp.full((_L,), bp, jnp.int32)], vals)
                return carry
            lax.fori_loop(0, _CHUNK, tbody, 0)

            for fg in range(_FT):
                for btl in range(_BT):
                    pltpu.async_copy(
                        tbuf_s.at[buf, pl.ds(8 * fg, 8),
                                  pl.ds(128 * btl, 128)],
                        out_hbm.at[i_t, fg, bc_t * _BT + btl], wsem_cur)

            @pl.when(t < _PER_W - 2)
            def _():
                start_idx_dma(t + 2, buf)

        def body(u, carry):
            one_item(2 * u, 0, gsem0, gsem1, wsem0)
            one_item(2 * u + 1, 1, gsem1, gsem0, wsem1)
            return carry

        lax.fori_loop(0, _PER_W // 2, body, 0)

        def wdrain2(j, carry):
            pltpu.make_async_copy(
                out_hbm.at[0, 0, 0],
                tbuf_s.at[0, pl.ds(0, 8), pl.ds(0, 128)], wsem0).wait()
            pltpu.make_async_copy(
                out_hbm.at[0, 0, 0],
                tbuf_s.at[1, pl.ds(0, 8), pl.ds(0, 128)], wsem1).wait()
            return carry
        lax.fori_loop(0, _FT * _BT, wdrain2, 0)

    return k


def kernel(x, table):
    V, D = table.shape
    xt = jnp.transpose(x).astype(jnp.int32)
    out5 = _make(V)(xt, table)
    return out5.transpose(2, 4, 0, 1, 3).reshape(_BB, _I, D)


# speedup vs baseline: 2.4315x; 1.0309x over previous
import functools

import jax
import jax.numpy as jnp
from jax import lax
from jax.experimental import pallas as pl
from jax.experimental.pallas import tpu as pltpu
from jax.experimental.pallas import tpu_sc as plsc

_NC = 2
_NS = 16
_NW = _NC * _NS
_L = 16

_I = 50
_BB = 16384
_D = 32
_CHUNK = 512
_BC = _BB // _CHUNK
_N_ITEMS = _I * _BC
_PER_W = _N_ITEMS // _NW
_FT = _D // 8
_BT = _CHUNK // 128


@functools.lru_cache(maxsize=None)
def _make(V: int):
    mesh = plsc.VectorSubcoreMesh(core_axis_name="c", subcore_axis_name="s")

    @functools.partial(
        pl.kernel,
        mesh=mesh,
        out_type=jax.ShapeDtypeStruct((_I, _FT, _BB // 128, 8, 128),
                                      jnp.float32),
        scratch_types=[
            pltpu.VMEM((2, _CHUNK), jnp.int32),
            pltpu.VMEM((2, _CHUNK, _D), jnp.float32),
            pltpu.VMEM((2, _D, _CHUNK + 1), jnp.float32),
            pltpu.SemaphoreType.DMA,
            pltpu.SemaphoreType.DMA,
            pltpu.SemaphoreType.DMA,
            pltpu.SemaphoreType.DMA,
            pltpu.SemaphoreType.DMA,
        ],
        compiler_params=pltpu.CompilerParams(use_tc_tiling_on_sc=False,
                                             needs_layout_passes=False),
    )
    def k(xt_hbm, tbl_hbm, out_hbm, idx_s, rows_s, tbuf_s,
          isem, gsem0, gsem1, wsem0, wsem1):
        wid = lax.axis_index("s") * _NC + lax.axis_index("c")
        g0 = wid * _PER_W

        def item(t):
            g = g0 + t
            return g // _BC, g % _BC

        def start_idx_dma(t, buf):
            i, bc = item(t)
            return pltpu.async_copy(
                xt_hbm.at[i, pl.ds(bc * _CHUNK, _CHUNK)], idx_s.at[buf], isem)

        def add_one(buf):
            def body(j, carry):
                sl = pl.ds(j * _L, _L)
                idx_s[buf, sl] = idx_s[buf, sl] + 1
                return carry
            lax.fori_loop(0, _CHUNK // _L, body, 0)

        def start_gather(buf, gsem):
            return pltpu.async_copy(
                tbl_hbm.at[idx_s.at[buf]], rows_s.at[buf], gsem)

        start_idx_dma(0, 0).wait()
        add_one(0)
        start_gather(0, gsem0)
        start_idx_dma(1, 1)

        iota16 = lax.iota(jnp.int32, _L)

        def one_item(t, buf, gsem_cur, gsem_nxt, wsem_cur):
            i_t, bc_t = item(t)

            @pl.when(t < _PER_W - 1)
            def _():
                pltpu.make_async_copy(
                    xt_hbm.at[0, pl.ds(0, _CHUNK)], idx_s.at[1 - buf],
                    isem).wait()
                add_one(1 - buf)
                start_gather(1 - buf, gsem_nxt)

            pltpu.make_async_copy(
                tbl_hbm.at[pl.ds(0, _CHUNK)], rows_s.at[buf],
                gsem_cur).wait()

            @pl.when(t >= 2)
            def _():
                def wdrain(j, carry):
                    pltpu.make_async_copy(
                        out_hbm.at[0, 0, 0],
                        tbuf_s.at[buf, pl.ds(0, 8), pl.ds(0, 128)],
                        wsem_cur).wait()
                    return carry
                lax.fori_loop(0, _FT * _BT, wdrain, 0)

            def tbody(j, carry):
                for u in range(4):
                    bp = 4 * j + u
                    for f0 in range(0, _D, _L):
                        vals = rows_s[buf, bp, pl.ds(f0, _L)]
                        plsc.store_scatter(
                            tbuf_s.at[buf],
                            [f0 + iota16, jnp.full((_L,), bp, jnp.int32)],
                            vals)
                return carry
            lax.fori_loop(0, _CHUNK // 4, tbody, 0)

            for fg in range(_FT):
                for btl in range(_BT):
                    pltpu.async_copy(
                        tbuf_s.at[buf, pl.ds(8 * fg, 8),
                                  pl.ds(128 * btl, 128)],
                        out_hbm.at[i_t, fg, bc_t * _BT + btl], wsem_cur)

            @pl.when(t < _PER_W - 2)
            def _():
                start_idx_dma(t + 2, buf)

        def body(u, carry):
            one_item(2 * u, 0, gsem0, gsem1, wsem0)
            one_item(2 * u + 1, 1, gsem1, gsem0, wsem1)
            return carry

        lax.fori_loop(0, _PER_W // 2, body, 0)

        def wdrain2(j, carry):
            pltpu.make_async_copy(
                out_hbm.at[0, 0, 0],
                tbuf_s.at[0, pl.ds(0, 8), pl.ds(0, 128)], wsem0).wait()
            pltpu.make_async_copy(
                out_hbm.at[0, 0, 0],
                tbuf_s.at[1, pl.ds(0, 8), pl.ds(0, 128)], wsem1).wait()
            return carry
        lax.fori_loop(0, _FT * _BT, wdrain2, 0)

    return k


def kernel(x, table):
    V, D = table.shape
    xt = jnp.transpose(x).astype(jnp.int32)
    out5 = _make(V)(xt, table)
    return out5.transpose(2, 4, 0, 1, 3).reshape(_BB, _I, D)


# speedup vs baseline: 2.4519x; 1.0084x over previous
import functools

import jax
import jax.numpy as jnp
from jax import lax
from jax.experimental import pallas as pl
from jax.experimental.pallas import tpu as pltpu
from jax.experimental.pallas import tpu_sc as plsc

_NC = 2
_NS = 16
_NW = _NC * _NS
_L = 16

_I = 50
_BB = 16384
_D = 32
_CHUNK = 512
_BC = _BB // _CHUNK
_N_ITEMS = _I * _BC
_PER_W = _N_ITEMS // _NW
_FT = _D // 8
_BT = _CHUNK // 128


@functools.lru_cache(maxsize=None)
def _make(V: int):
    mesh = plsc.VectorSubcoreMesh(core_axis_name="c", subcore_axis_name="s")

    @functools.partial(
        pl.kernel,
        mesh=mesh,
        out_type=jax.ShapeDtypeStruct((_I, _FT, _BB // 128, 8, 128),
                                      jnp.float32),
        scratch_types=[
            pltpu.VMEM((2, _CHUNK), jnp.int32),
            pltpu.VMEM((2, _CHUNK, _D), jnp.float32),
            pltpu.VMEM((2, _D, _CHUNK + 1), jnp.float32),
            pltpu.SemaphoreType.DMA,
            pltpu.SemaphoreType.DMA,
            pltpu.SemaphoreType.DMA,
            pltpu.SemaphoreType.DMA,
            pltpu.SemaphoreType.DMA,
        ],
        compiler_params=pltpu.CompilerParams(use_tc_tiling_on_sc=False,
                                             needs_layout_passes=False),
    )
    def k(xt_hbm, tbl_hbm, out_hbm, idx_s, rows_s, tbuf_s,
          isem, gsem0, gsem1, wsem0, wsem1):
        wid = lax.axis_index("s") * _NC + lax.axis_index("c")
        g0 = wid * _PER_W

        def item(t):
            g = g0 + t
            return g // _BC, g % _BC

        def start_idx_dma(t, buf):
            i, bc = item(t)
            return pltpu.async_copy(
                xt_hbm.at[i, pl.ds(bc * _CHUNK, _CHUNK)], idx_s.at[buf], isem)

        def add_one(buf):
            def body(j, carry):
                sl = pl.ds(j * _L, _L)
                idx_s[buf, sl] = idx_s[buf, sl] + 1
                return carry
            lax.fori_loop(0, _CHUNK // _L, body, 0)

        def start_gather(buf, gsem):
            return pltpu.async_copy(
                tbl_hbm.at[idx_s.at[buf]], rows_s.at[buf], gsem)

        start_idx_dma(0, 0).wait()
        add_one(0)
        start_gather(0, gsem0)
        start_idx_dma(1, 1)

        iota16 = lax.iota(jnp.int32, _L)

        def one_item(t, buf, gsem_cur, gsem_nxt, wsem_cur):
            i_t, bc_t = item(t)

            @pl.when(t < _PER_W - 1)
            def _():
                pltpu.make_async_copy(
                    xt_hbm.at[0, pl.ds(0, _CHUNK)], idx_s.at[1 - buf],
                    isem).wait()
                add_one(1 - buf)
                start_gather(1 - buf, gsem_nxt)

            pltpu.make_async_copy(
                tbl_hbm.at[pl.ds(0, _CHUNK)], rows_s.at[buf],
                gsem_cur).wait()

            @pl.when(t >= 2)
            def _():
                def wdrain(j, carry):
                    pltpu.make_async_copy(
                        out_hbm.at[0, 0, 0],
                        tbuf_s.at[buf, pl.ds(0, 8), pl.ds(0, 128)],
                        wsem_cur).wait()
                    return carry
                lax.fori_loop(0, _FT * _BT, wdrain, 0)

            for f0 in range(0, _D, _L):
                def tbody(j, carry, f0=f0):
                    for u in range(4):
                        bp = 4 * j + u
                        vals = rows_s[buf, bp, pl.ds(f0, _L)]
                        plsc.store_scatter(
                            tbuf_s.at[buf],
                            [f0 + iota16, jnp.full((_L,), bp, jnp.int32)],
                            vals)
                    return carry
                lax.fori_loop(0, _CHUNK // 4, tbody, 0)
                for fg in (f0 // 8, f0 // 8 + 1):
                    for btl in range(_BT):
                        pltpu.async_copy(
                            tbuf_s.at[buf, pl.ds(8 * fg, 8),
                                      pl.ds(128 * btl, 128)],
                            out_hbm.at[i_t, fg, bc_t * _BT + btl], wsem_cur)

            @pl.when(t < _PER_W - 2)
            def _():
                start_idx_dma(t + 2, buf)

        def body(u, carry):
            one_item(2 * u, 0, gsem0, gsem1, wsem0)
            one_item(2 * u + 1, 1, gsem1, gsem0, wsem1)
            return carry

        lax.fori_loop(0, _PER_W // 2, body, 0)

        def wdrain2(j, carry):
            pltpu.make_async_copy(
                out_hbm.at[0, 0, 0],
                tbuf_s.at[0, pl.ds(0, 8), pl.ds(0, 128)], wsem0).wait()
            pltpu.make_async_copy(
                out_hbm.at[0, 0, 0],
                tbuf_s.at[1, pl.ds(0, 8), pl.ds(0, 128)], wsem1).wait()
            return carry
        lax.fori_loop(0, _FT * _BT, wdrain2, 0)

    return k


def kernel(x, table):
    V, D = table.shape
    xt = jnp.transpose(x).astype(jnp.int32)
    out5 = _make(V)(xt, table)
    return out5.transpose(2, 4, 0, 1, 3).reshape(_BB, _I, D)
